# baseline (speedup 1.0000x reference)
import functools

import jax
import jax.numpy as jnp
from jax import lax
from jax.experimental import pallas as pl
from jax.experimental.pallas import tpu as pltpu
from jax.experimental.pallas import tpu_sc as plsc

_NUM_CLASSES = 80
_CONF_THR = 0.05
_IOU_THR = 0.5
_MAX_PER_CLASS = 100
_MAX_DET = 100
_PRE_TOPK = 1000

_A = 49104
_ROWS = 320
_L = 16
_VECS = _A // _L
_GT_CAP = 1024
_EQ_CAP = 1024
_WIDTH = _GT_CAP + _EQ_CAP
_NWORKERS = 32
_ROWS_PER = _ROWS // _NWORKERS


def _sc_select_body(scores_hbm, cs_hbm, ci_hbm, data_v, hist_v, bufs_v, bufi_v):
    wid = lax.axis_index("s") * 2 + lax.axis_index("c")
    lane = lax.iota(jnp.int32, _L)
    ones_i = jnp.ones((_L,), jnp.int32)

    def row_body(ri, _carry):
        r = wid * _ROWS_PER + ri
        pltpu.sync_copy(scores_hbm.at[r], data_v)

        prefix = jnp.int32(0)
        c_above = jnp.int32(0)
        for p in range(4):
            shift = 24 - 8 * p

            def zero_body(i, c):
                hist_v[pl.ds(i * _L, _L)] = jnp.zeros((_L,), jnp.int32)
                return c
            lax.fori_loop(0, 256, zero_body, 0)

            def hist_body(i, c, _shift=shift, _prefix=prefix):
                v = data_v[pl.ds(i * _L, _L)]
                b = lax.bitcast_convert_type(v, jnp.int32)
                key = lax.shift_right_logical(b, _shift)
                binv = lax.bitwise_and(key, 0xFF)
                match = lax.shift_right_logical(key, 8) == _prefix
                idx = lax.bitwise_or(lax.shift_left(binv, 4), lane)
                plsc.addupdate_scatter(hist_v, [idx], ones_i, mask=match)
                return c
            lax.fori_loop(0, _VECS, hist_body, 0)

            def scan_body(i, carry, _c_above=c_above):
                cum, found_bin, c_add = carry
                bnum = 255 - i
                t = jnp.sum(hist_v[pl.ds(bnum * _L, _L)])
                not_found = found_bin < 0
                hit = not_found & (_c_above + cum + t >= _PRE_TOPK)
                found_bin = jnp.where(hit, bnum, found_bin)
                c_add = jnp.where(hit, cum, c_add)
                cum = jnp.where(not_found & jnp.logical_not(hit), cum + t, cum)
                return cum, found_bin, c_add
            _, fbin, c_add = lax.fori_loop(
                0, 256, scan_body, (jnp.int32(0), jnp.int32(-1), jnp.int32(0)))
            prefix = lax.bitwise_or(lax.shift_left(prefix, 8), fbin)
            c_above = c_above + c_add

        thr_bits = prefix
        need_ties = _PRE_TOPK - c_above

        def init_body(i, c):
            bufs_v[pl.ds(i * _L, _L)] = jnp.full((_L,), -1.0, jnp.float32)
            bufi_v[pl.ds(i * _L, _L)] = jnp.zeros((_L,), jnp.int32)
            return c
        lax.fori_loop(0, _WIDTH // _L, init_body, 0)

        def comp_body(i, carry):
            cgt, ceq = carry
            v = data_v[pl.ds(i * _L, _L)]
            b = lax.bitcast_convert_type(v, jnp.int32)
            gidx = lane + i * _L
            m_gt = b > thr_bits
            m_eq = b == thr_bits
            plsc.store_compressed(bufs_v.at[pl.ds(cgt, _L)], v, mask=m_gt)
            plsc.store_compressed(bufi_v.at[pl.ds(cgt, _L)], gidx, mask=m_gt)
            cgt = cgt + jnp.sum(m_gt.astype(jnp.int32))

            @pl.when(ceq <= _EQ_CAP - _L)
            def _():
                plsc.store_compressed(
                    bufs_v.at[pl.ds(_GT_CAP + ceq, _L)], v, mask=m_eq)
                plsc.store_compressed(
                    bufi_v.at[pl.ds(_GT_CAP + ceq, _L)], gidx, mask=m_eq)
            ceq = jnp.minimum(ceq + jnp.sum(m_eq.astype(jnp.int32)),
                              jnp.int32(_EQ_CAP))
            return cgt, ceq
        lax.fori_loop(0, _VECS, comp_body, (jnp.int32(0), jnp.int32(0)))

        def trim_body(i, c):
            pos = lane + i * _L
            v = bufs_v[pl.ds(_GT_CAP + i * _L, _L)]
            bufs_v[pl.ds(_GT_CAP + i * _L, _L)] = jnp.where(
                pos < need_ties, v, -1.0)
            return c
        lax.fori_loop(0, _EQ_CAP // _L, trim_body, 0)

        pltpu.sync_copy(bufs_v, cs_hbm.at[r])
        pltpu.sync_copy(bufi_v, ci_hbm.at[r])
        return _carry

    lax.fori_loop(0, _ROWS_PER, row_body, 0)


def _sc_select(scores_rows):
    mesh = plsc.VectorSubcoreMesh(core_axis_name="c", subcore_axis_name="s")
    return pl.kernel(
        _sc_select_body,
        mesh=mesh,
        out_type=(
            jax.ShapeDtypeStruct((_ROWS, _WIDTH), jnp.float32),
            jax.ShapeDtypeStruct((_ROWS, _WIDTH), jnp.int32),
        ),
        scratch_types=[
            pltpu.VMEM((_A,), jnp.float32),
            pltpu.VMEM((256 * _L,), jnp.int32),
            pltpu.VMEM((_WIDTH,), jnp.float32),
            pltpu.VMEM((_WIDTH,), jnp.int32),
        ],
        compiler_params=pltpu.CompilerParams(needs_layout_passes=False),
    )(scores_rows)


_OUT_LANES = 128


def _nms_body(sc_ref, x1_ref, y1_ref, x2_ref, y2_ref,
              osc_ref, ox1_ref, oy1_ref, ox2_ref, oy2_ref):
    rows, lanes = sc_ref.shape
    sc0 = sc_ref[:]
    x1 = x1_ref[:]
    y1 = y1_ref[:]
    x2 = x2_ref[:]
    y2 = y2_ref[:]
    sc0 = jnp.where(sc0 > _CONF_THR, sc0, -1.0)
    a2 = (x2 - x1) * (y2 - y1)
    lane_iota = jax.lax.broadcasted_iota(jnp.int32, (rows, lanes), 1)
    col_iota = jax.lax.broadcasted_iota(jnp.int32, (rows, _OUT_LANES), 1)
    zeros_out = jnp.zeros((rows, _OUT_LANES), jnp.float32)

    def step(t, carry):
        sc, osc, ox1, oy1, ox2, oy2 = carry
        m = jnp.max(sc, axis=1, keepdims=True)
        keep = m > 0.0
        j = jnp.min(jnp.where(sc == m, lane_iota, jnp.int32(1 << 30)),
                    axis=1, keepdims=True)
        oh = lane_iota == j
        bx1 = jnp.sum(jnp.where(oh, x1, 0.0), axis=1, keepdims=True)
        by1 = jnp.sum(jnp.where(oh, y1, 0.0), axis=1, keepdims=True)
        bx2 = jnp.sum(jnp.where(oh, x2, 0.0), axis=1, keepdims=True)
        by2 = jnp.sum(jnp.where(oh, y2, 0.0), axis=1, keepdims=True)
        a1 = (bx2 - bx1) * (by2 - by1)
        ltx = jnp.maximum(bx1, x1)
        lty = jnp.maximum(by1, y1)
        rbx = jnp.minimum(bx2, x2)
        rby = jnp.minimum(by2, y2)
        w = jnp.maximum(rbx - ltx, 0.0)
        h = jnp.maximum(rby - lty, 0.0)
        inter = w * h
        iou = inter / (a1 + a2 - inter + 1e-8)
        supp = (iou > _IOU_THR) | oh
        sc = jnp.where(supp, -1.0, sc)
        col = col_iota == t
        osc = osc + jnp.where(col, jnp.where(keep, m, -1.0), 0.0)
        ox1 = ox1 + jnp.where(col, jnp.where(keep, bx1, 0.0), 0.0)
        oy1 = oy1 + jnp.where(col, jnp.where(keep, by1, 0.0), 0.0)
        ox2 = ox2 + jnp.where(col, jnp.where(keep, bx2, 0.0), 0.0)
        oy2 = oy2 + jnp.where(col, jnp.where(keep, by2, 0.0), 0.0)
        return sc, osc, ox1, oy1, ox2, oy2

    carry = (sc0, zeros_out, zeros_out, zeros_out, zeros_out, zeros_out)
    _, osc, ox1, oy1, ox2, oy2 = jax.lax.fori_loop(
        0, _MAX_PER_CLASS, step, carry)
    osc_ref[:] = osc
    ox1_ref[:] = ox1
    oy1_ref[:] = oy1
    ox2_ref[:] = ox2
    oy2_ref[:] = oy2


def _run_nms(scores, bx1, by1, bx2, by2):
    rows = scores.shape[0]
    out_sds = jax.ShapeDtypeStruct((rows, _OUT_LANES), jnp.float32)
    return pl.pallas_call(
        _nms_body,
        out_shape=(out_sds,) * 5,
    )(scores, bx1, by1, bx2, by2)


def _anchor_dims_np():
    import numpy as np
    ratios = [0.5, 1.0, 2.0]
    scales = [2.0 ** 0.0, 2.0 ** (1.0 / 3.0), 2.0 ** (2.0 / 3.0)]
    dims_all = []
    for area in [32.0 ** 2, 64.0 ** 2, 128.0 ** 2, 256.0 ** 2, 512.0 ** 2]:
        dims = []
        for r in ratios:
            h = np.sqrt(area / r)
            w = area / h
            for s in scales:
                dims.append([s * w, s * h])
        dims_all.append(np.array(dims, np.float32))
    return dims_all


def _get_anchors_np(H, W):
    import numpy as np
    strides = [2 ** i for i in range(3, 8)]
    dims_all = _anchor_dims_np()
    out = []
    for lvl in range(5):
        fh = int(np.ceil(H / strides[lvl]))
        fw = int(np.ceil(W / strides[lvl]))
        rx = (np.arange(fw, dtype=np.float32) + 0.5) * strides[lvl]
        ry = (np.arange(fh, dtype=np.float32) + 0.5) * strides[lvl]
        cx, cy = np.meshgrid(rx, ry)
        centers = np.tile(np.stack([cx, cy], -1)[:, :, None, :], [1, 1, 9, 1])
        dims = np.tile(dims_all[lvl][None, None, :, :], [fh, fw, 1, 1])
        out.append(np.concatenate([centers, dims], -1).reshape(-1, 4))
    return np.concatenate(out, 0)


@jax.jit
def kernel(images, predictions):
    H, W = images.shape[1], images.shape[2]
    anchors = jnp.asarray(_get_anchors_np(H, W))
    B = predictions.shape[0]

    box_preds = predictions[..., :4]
    cls_scores = jax.nn.sigmoid(predictions[..., 4:])
    sc_rows_full = jnp.transpose(cls_scores, (0, 2, 1)).reshape(_ROWS, _A)

    cand_s, cand_i = _sc_select(sc_rows_full)

    top_idx = cand_i.reshape(B, _NUM_CLASSES, _WIDTH)
    bp = jnp.take_along_axis(box_preds[:, None, :, :],
                             top_idx[..., None], axis=2)
    an = anchors[top_idx]
    bvar = jnp.asarray([0.1, 0.1, 0.2, 0.2], jnp.float32)
    b = bp * bvar
    cxcy = b[..., :2] * an[..., 2:] + an[..., :2]
    wh = jnp.exp(b[..., 2:]) * an[..., 2:]
    boxes = jnp.concatenate([cxcy - wh / 2.0, cxcy + wh / 2.0], axis=-1)

    coords = boxes.reshape(_ROWS, _WIDTH, 4)
    bx1 = coords[:, :, 0]
    by1 = coords[:, :, 1]
    bx2 = coords[:, :, 2]
    by2 = coords[:, :, 3]

    osc, ox1, oy1, ox2, oy2 = _run_nms(cand_s, bx1, by1, bx2, by2)

    fs = osc[:, :_MAX_PER_CLASS].reshape(B, -1)
    fb = jnp.stack([ox1, oy1, ox2, oy2], axis=-1)[:, :_MAX_PER_CLASS, :]
    fb = fb.reshape(B, _NUM_CLASSES * _MAX_PER_CLASS, 4)

    ts, ti = jax.lax.top_k(fs, _MAX_DET)
    sel_b = jnp.take_along_axis(fb, ti[..., None], axis=1)
    sel_c = (ti // _MAX_PER_CLASS).astype(jnp.float32)
    mask = ts > 0.0
    ts_out = jnp.where(mask, ts, 0.0)
    sel_b = jnp.where(mask[..., None], sel_b, 0.0)
    sel_c = jnp.where(mask, sel_c, 0.0)
    valid = jnp.sum(mask.astype(jnp.int32), axis=1)
    return sel_b, ts_out, sel_c, valid


# speedup vs baseline: 1.0545x; 1.0545x over previous
import functools

import jax
import jax.numpy as jnp
from jax import lax
from jax.experimental import pallas as pl
from jax.experimental.pallas import tpu as pltpu
from jax.experimental.pallas import tpu_sc as plsc

_NUM_CLASSES = 80
_CONF_THR = 0.05
_IOU_THR = 0.5
_MAX_PER_CLASS = 100
_MAX_DET = 100
_PRE_TOPK = 1000

_A = 49104
_ROWS = 320
_L = 16
_VECS = _A // _L
_GT_CAP = 1024
_EQ_CAP = 1024
_WIDTH = _GT_CAP + _EQ_CAP
_NWORKERS = 32
_ROWS_PER = _ROWS // _NWORKERS


def _sc_select_body(scores_hbm, cs_hbm, ci_hbm,
                    data_v, hist_v, bufs_v, bufi_v):
    wid = lax.axis_index("s") * 2 + lax.axis_index("c")
    lane = lax.iota(jnp.int32, _L)
    ones_i = jnp.ones((_L,), jnp.int32)

    def row_body(ri, _carry):
        r = wid * _ROWS_PER + ri
        img_base = (r // _NUM_CLASSES) * _A
        pltpu.sync_copy(scores_hbm.at[r], data_v)

        prefix = jnp.int32(0)
        c_above = jnp.int32(0)
        for p in range(4):
            shift = 24 - 8 * p

            def zero_body(i, c):
                hist_v[pl.ds(i * _L, _L)] = jnp.zeros((_L,), jnp.int32)
                return c
            lax.fori_loop(0, 256, zero_body, 0)

            def hist_body(i, c, _shift=shift, _prefix=prefix):
                v = data_v[pl.ds(i * _L, _L)]
                b = lax.bitcast_convert_type(v, jnp.int32)
                key = lax.shift_right_logical(b, _shift)
                binv = lax.bitwise_and(key, 0xFF)
                match = lax.shift_right_logical(key, 8) == _prefix
                idx = lax.bitwise_or(lax.shift_left(binv, 4), lane)
                plsc.addupdate_scatter(hist_v, [idx], ones_i, mask=match)
                return c
            lax.fori_loop(0, _VECS, hist_body, 0)

            def scan_body(i, carry, _c_above=c_above):
                cum, found_bin, c_add = carry
                bnum = 255 - i
                t = jnp.sum(hist_v[pl.ds(bnum * _L, _L)])
                not_found = found_bin < 0
                hit = not_found & (_c_above + cum + t >= _PRE_TOPK)
                found_bin = jnp.where(hit, bnum, found_bin)
                c_add = jnp.where(hit, cum, c_add)
                cum = jnp.where(not_found & jnp.logical_not(hit), cum + t, cum)
                return cum, found_bin, c_add
            _, fbin, c_add = lax.fori_loop(
                0, 256, scan_body, (jnp.int32(0), jnp.int32(-1), jnp.int32(0)))
            prefix = lax.bitwise_or(lax.shift_left(prefix, 8), fbin)
            c_above = c_above + c_add

        thr_bits = prefix
        need_ties = _PRE_TOPK - c_above

        def init_body(i, c):
            bufs_v[pl.ds(i * _L, _L)] = jnp.full((_L,), -1.0, jnp.float32)
            bufi_v[pl.ds(i * _L, _L)] = jnp.full((_L,), img_base, jnp.int32)
            return c
        lax.fori_loop(0, _WIDTH // _L, init_body, 0)

        def comp_body(i, carry):
            cgt, ceq = carry
            v = data_v[pl.ds(i * _L, _L)]
            b = lax.bitcast_convert_type(v, jnp.int32)
            gidx = lane + (img_base + i * _L)
            m_gt = b > thr_bits
            m_eq = b == thr_bits
            plsc.store_compressed(bufs_v.at[pl.ds(cgt, _L)], v, mask=m_gt)
            plsc.store_compressed(bufi_v.at[pl.ds(cgt, _L)], gidx, mask=m_gt)
            cgt = cgt + jnp.sum(m_gt.astype(jnp.int32))

            @pl.when(ceq <= _EQ_CAP - _L)
            def _():
                plsc.store_compressed(
                    bufs_v.at[pl.ds(_GT_CAP + ceq, _L)], v, mask=m_eq)
                plsc.store_compressed(
                    bufi_v.at[pl.ds(_GT_CAP + ceq, _L)], gidx, mask=m_eq)
            ceq = jnp.minimum(ceq + jnp.sum(m_eq.astype(jnp.int32)),
                              jnp.int32(_EQ_CAP))
            return cgt, ceq
        lax.fori_loop(0, _VECS, comp_body, (jnp.int32(0), jnp.int32(0)))

        def trim_body(i, c):
            pos = lane + i * _L
            v = bufs_v[pl.ds(_GT_CAP + i * _L, _L)]
            bufs_v[pl.ds(_GT_CAP + i * _L, _L)] = jnp.where(
                pos < need_ties, v, -1.0)
            return c
        lax.fori_loop(0, _EQ_CAP // _L, trim_body, 0)

        pltpu.sync_copy(bufs_v, cs_hbm.at[r])
        pltpu.sync_copy(bufi_v, ci_hbm.at[r])
        return _carry

    lax.fori_loop(0, _ROWS_PER, row_body, 0)


def _sc_select(scores_rows):
    mesh = plsc.VectorSubcoreMesh(core_axis_name="c", subcore_axis_name="s")
    return pl.kernel(
        _sc_select_body,
        mesh=mesh,
        out_type=(
            jax.ShapeDtypeStruct((_ROWS, _WIDTH), jnp.float32),
            jax.ShapeDtypeStruct((_ROWS, _WIDTH), jnp.int32),
        ),
        scratch_types=[
            pltpu.VMEM((_A,), jnp.float32),
            pltpu.VMEM((256 * _L,), jnp.int32),
            pltpu.VMEM((_WIDTH,), jnp.float32),
            pltpu.VMEM((_WIDTH,), jnp.int32),
        ],
        compiler_params=pltpu.CompilerParams(needs_layout_passes=False),
    )(scores_rows)


_OUT_LANES = 128


def _nms_body(sc_ref, x1_ref, y1_ref, x2_ref, y2_ref,
              osc_ref, ox1_ref, oy1_ref, ox2_ref, oy2_ref):
    rows, lanes = sc_ref.shape
    sc0 = sc_ref[:]
    x1 = x1_ref[:]
    y1 = y1_ref[:]
    x2 = x2_ref[:]
    y2 = y2_ref[:]
    sc0 = jnp.where(sc0 > _CONF_THR, sc0, -1.0)
    a2 = (x2 - x1) * (y2 - y1)
    lane_iota = jax.lax.broadcasted_iota(jnp.int32, (rows, lanes), 1)
    col_iota = jax.lax.broadcasted_iota(jnp.int32, (rows, _OUT_LANES), 1)
    zeros_out = jnp.zeros((rows, _OUT_LANES), jnp.float32)

    def step(t, carry):
        sc, osc, ox1, oy1, ox2, oy2 = carry
        m = jnp.max(sc, axis=1, keepdims=True)
        keep = m > 0.0
        j = jnp.min(jnp.where(sc == m, lane_iota, jnp.int32(1 << 30)),
                    axis=1, keepdims=True)
        oh = lane_iota == j
        bx1 = jnp.sum(jnp.where(oh, x1, 0.0), axis=1, keepdims=True)
        by1 = jnp.sum(jnp.where(oh, y1, 0.0), axis=1, keepdims=True)
        bx2 = jnp.sum(jnp.where(oh, x2, 0.0), axis=1, keepdims=True)
        by2 = jnp.sum(jnp.where(oh, y2, 0.0), axis=1, keepdims=True)
        a1 = (bx2 - bx1) * (by2 - by1)
        ltx = jnp.maximum(bx1, x1)
        lty = jnp.maximum(by1, y1)
        rbx = jnp.minimum(bx2, x2)
        rby = jnp.minimum(by2, y2)
        w = jnp.maximum(rbx - ltx, 0.0)
        h = jnp.maximum(rby - lty, 0.0)
        inter = w * h
        iou = inter / (a1 + a2 - inter + 1e-8)
        supp = (iou > _IOU_THR) | oh
        sc = jnp.where(supp, -1.0, sc)
        col = col_iota == t
        osc = osc + jnp.where(col, jnp.where(keep, m, -1.0), 0.0)
        ox1 = ox1 + jnp.where(col, jnp.where(keep, bx1, 0.0), 0.0)
        oy1 = oy1 + jnp.where(col, jnp.where(keep, by1, 0.0), 0.0)
        ox2 = ox2 + jnp.where(col, jnp.where(keep, bx2, 0.0), 0.0)
        oy2 = oy2 + jnp.where(col, jnp.where(keep, by2, 0.0), 0.0)
        return sc, osc, ox1, oy1, ox2, oy2

    carry = (sc0, zeros_out, zeros_out, zeros_out, zeros_out, zeros_out)
    _, osc, ox1, oy1, ox2, oy2 = jax.lax.fori_loop(
        0, _MAX_PER_CLASS, step, carry)
    osc_ref[:] = osc
    ox1_ref[:] = ox1
    oy1_ref[:] = oy1
    ox2_ref[:] = ox2
    oy2_ref[:] = oy2


def _run_nms(scores, bx1, by1, bx2, by2):
    rows = scores.shape[0]
    out_sds = jax.ShapeDtypeStruct((rows, _OUT_LANES), jnp.float32)
    return pl.pallas_call(
        _nms_body,
        out_shape=(out_sds,) * 5,
    )(scores, bx1, by1, bx2, by2)


def _anchor_dims_np():
    import numpy as np
    ratios = [0.5, 1.0, 2.0]
    scales = [2.0 ** 0.0, 2.0 ** (1.0 / 3.0), 2.0 ** (2.0 / 3.0)]
    dims_all = []
    for area in [32.0 ** 2, 64.0 ** 2, 128.0 ** 2, 256.0 ** 2, 512.0 ** 2]:
        dims = []
        for r in ratios:
            h = np.sqrt(area / r)
            w = area / h
            for s in scales:
                dims.append([s * w, s * h])
        dims_all.append(np.array(dims, np.float32))
    return dims_all


def _get_anchors_np(H, W):
    import numpy as np
    strides = [2 ** i for i in range(3, 8)]
    dims_all = _anchor_dims_np()
    out = []
    for lvl in range(5):
        fh = int(np.ceil(H / strides[lvl]))
        fw = int(np.ceil(W / strides[lvl]))
        rx = (np.arange(fw, dtype=np.float32) + 0.5) * strides[lvl]
        ry = (np.arange(fh, dtype=np.float32) + 0.5) * strides[lvl]
        cx, cy = np.meshgrid(rx, ry)
        centers = np.tile(np.stack([cx, cy], -1)[:, :, None, :], [1, 1, 9, 1])
        dims = np.tile(dims_all[lvl][None, None, :, :], [fh, fw, 1, 1])
        out.append(np.concatenate([centers, dims], -1).reshape(-1, 4))
    return np.concatenate(out, 0)


@jax.jit
def kernel(images, predictions):
    H, W = images.shape[1], images.shape[2]
    anchors = jnp.asarray(_get_anchors_np(H, W))
    B = predictions.shape[0]

    box_preds = predictions[..., :4]
    cls_scores = jax.nn.sigmoid(predictions[..., 4:])
    sc_rows_full = jnp.transpose(cls_scores, (0, 2, 1)).reshape(_ROWS, _A)

    table = jnp.concatenate(
        [box_preds, jnp.broadcast_to(anchors[None], (B, _A, 4))],
        axis=-1).reshape(B * _A, 8)

    cand_s, cand_i = _sc_select(sc_rows_full)

    cand_bp = jnp.take(table, cand_i.reshape(-1), axis=0)
    bp = cand_bp[..., :4].reshape(B, _NUM_CLASSES, _WIDTH, 4)
    an = cand_bp[..., 4:].reshape(B, _NUM_CLASSES, _WIDTH, 4)
    bvar = jnp.asarray([0.1, 0.1, 0.2, 0.2], jnp.float32)
    b = bp * bvar
    cxcy = b[..., :2] * an[..., 2:] + an[..., :2]
    wh = jnp.exp(b[..., 2:]) * an[..., 2:]
    boxes = jnp.concatenate([cxcy - wh / 2.0, cxcy + wh / 2.0], axis=-1)

    coords = boxes.reshape(_ROWS, _WIDTH, 4)
    bx1 = coords[:, :, 0]
    by1 = coords[:, :, 1]
    bx2 = coords[:, :, 2]
    by2 = coords[:, :, 3]

    osc, ox1, oy1, ox2, oy2 = _run_nms(cand_s, bx1, by1, bx2, by2)

    fs = osc[:, :_MAX_PER_CLASS].reshape(B, -1)
    fb = jnp.stack([ox1, oy1, ox2, oy2], axis=-1)[:, :_MAX_PER_CLASS, :]
    fb = fb.reshape(B, _NUM_CLASSES * _MAX_PER_CLASS, 4)

    ts, ti = jax.lax.top_k(fs, _MAX_DET)
    sel_b = jnp.take_along_axis(fb, ti[..., None], axis=1)
    sel_c = (ti // _MAX_PER_CLASS).astype(jnp.float32)
    mask = ts > 0.0
    ts_out = jnp.where(mask, ts, 0.0)
    sel_b = jnp.where(mask[..., None], sel_b, 0.0)
    sel_c = jnp.where(mask, sel_c, 0.0)
    valid = jnp.sum(mask.astype(jnp.int32), axis=1)
    return sel_b, ts_out, sel_c, valid


# speedup vs baseline: 1.0584x; 1.0037x over previous
import functools

import jax
import jax.numpy as jnp
from jax import lax
from jax.experimental import pallas as pl
from jax.experimental.pallas import tpu as pltpu
from jax.experimental.pallas import tpu_sc as plsc

_NUM_CLASSES = 80
_CONF_THR = 0.05
_IOU_THR = 0.5
_MAX_PER_CLASS = 100
_MAX_DET = 100
_PRE_TOPK = 1000

_A = 49104
_A_PAD = 49152
_ROWS = 320
_L = 16
_VECS = _A_PAD // _L
_GT_CAP = 1024
_EQ_CAP = 1024
_WIDTH = _GT_CAP + _EQ_CAP
_NWORKERS = 32
_ROWS_PER = _ROWS // _NWORKERS


def _sc_select_body(scores_hbm, cs_hbm, ci_hbm,
                    data_v, hist_v, bufs_v, bufi_v):
    wid = lax.axis_index("s") * 2 + lax.axis_index("c")
    lane = lax.iota(jnp.int32, _L)
    ones_i = jnp.ones((_L,), jnp.int32)

    def row_body(ri, _carry):
        r = wid * _ROWS_PER + ri
        img_base = (r // _NUM_CLASSES) * _A
        pltpu.sync_copy(scores_hbm.at[r], data_v)

        prefix = jnp.int32(0)
        c_above = jnp.int32(0)
        for p in range(4):
            shift = 24 - 8 * p

            def zero_body(i, c):
                hist_v[pl.ds(i * _L, _L)] = jnp.zeros((_L,), jnp.int32)
                return c
            lax.fori_loop(0, 256, zero_body, 0)

            def hist_body(i, c, _shift=shift, _prefix=prefix):
                v = data_v[i >> 3, pl.ds((i & 7) * _L, _L)]
                b = lax.bitcast_convert_type(v, jnp.int32)
                key = lax.shift_right_logical(b, _shift)
                binv = lax.bitwise_and(key, 0xFF)
                match = lax.shift_right_logical(key, 8) == _prefix
                idx = lax.bitwise_or(lax.shift_left(binv, 4), lane)
                plsc.addupdate_scatter(hist_v, [idx], ones_i, mask=match)
                return c
            lax.fori_loop(0, _VECS, hist_body, 0)

            def scan_body(i, carry, _c_above=c_above):
                cum, found_bin, c_add = carry
                bnum = 255 - i
                t = jnp.sum(hist_v[pl.ds(bnum * _L, _L)])
                not_found = found_bin < 0
                hit = not_found & (_c_above + cum + t >= _PRE_TOPK)
                found_bin = jnp.where(hit, bnum, found_bin)
                c_add = jnp.where(hit, cum, c_add)
                cum = jnp.where(not_found & jnp.logical_not(hit), cum + t, cum)
                return cum, found_bin, c_add
            _, fbin, c_add = lax.fori_loop(
                0, 256, scan_body, (jnp.int32(0), jnp.int32(-1), jnp.int32(0)))
            prefix = lax.bitwise_or(lax.shift_left(prefix, 8), fbin)
            c_above = c_above + c_add

        thr_bits = prefix
        need_ties = _PRE_TOPK - c_above

        def init_body(i, c):
            bufs_v[pl.ds(i * _L, _L)] = jnp.full((_L,), -1.0, jnp.float32)
            bufi_v[pl.ds(i * _L, _L)] = jnp.full((_L,), img_base, jnp.int32)
            return c
        lax.fori_loop(0, _WIDTH // _L, init_body, 0)

        def comp_body(i, carry):
            cgt, ceq = carry
            v = data_v[i >> 3, pl.ds((i & 7) * _L, _L)]
            b = lax.bitcast_convert_type(v, jnp.int32)
            gidx = lane + (img_base + i * _L)
            m_gt = b > thr_bits
            m_eq = b == thr_bits
            plsc.store_compressed(bufs_v.at[pl.ds(cgt, _L)], v, mask=m_gt)
            plsc.store_compressed(bufi_v.at[pl.ds(cgt, _L)], gidx, mask=m_gt)
            cgt = cgt + jnp.sum(m_gt.astype(jnp.int32))

            @pl.when(ceq <= _EQ_CAP - _L)
            def _():
                plsc.store_compressed(
                    bufs_v.at[pl.ds(_GT_CAP + ceq, _L)], v, mask=m_eq)
                plsc.store_compressed(
                    bufi_v.at[pl.ds(_GT_CAP + ceq, _L)], gidx, mask=m_eq)
            ceq = jnp.minimum(ceq + jnp.sum(m_eq.astype(jnp.int32)),
                              jnp.int32(_EQ_CAP))
            return cgt, ceq
        lax.fori_loop(0, _VECS, comp_body, (jnp.int32(0), jnp.int32(0)))

        def trim_body(i, c):
            pos = lane + i * _L
            v = bufs_v[pl.ds(_GT_CAP + i * _L, _L)]
            bufs_v[pl.ds(_GT_CAP + i * _L, _L)] = jnp.where(
                pos < need_ties, v, -1.0)
            return c
        lax.fori_loop(0, _EQ_CAP // _L, trim_body, 0)

        pltpu.sync_copy(bufs_v, cs_hbm.at[r])
        pltpu.sync_copy(bufi_v, ci_hbm.at[r])
        return _carry

    lax.fori_loop(0, _ROWS_PER, row_body, 0)


def _sc_select(scores_rows):
    mesh = plsc.VectorSubcoreMesh(core_axis_name="c", subcore_axis_name="s")
    return pl.kernel(
        _sc_select_body,
        mesh=mesh,
        out_type=(
            jax.ShapeDtypeStruct((_ROWS, _WIDTH), jnp.float32),
            jax.ShapeDtypeStruct((_ROWS, _WIDTH), jnp.int32),
        ),
        scratch_types=[
            pltpu.VMEM((_A_PAD // 128, 128), jnp.float32),
            pltpu.VMEM((256 * _L,), jnp.int32),
            pltpu.VMEM((_WIDTH,), jnp.float32),
            pltpu.VMEM((_WIDTH,), jnp.int32),
        ],
        compiler_params=pltpu.CompilerParams(needs_layout_passes=False),
    )(scores_rows)


_OUT_LANES = 128


def _nms_body(sc_ref, x1_ref, y1_ref, x2_ref, y2_ref,
              osc_ref, ox1_ref, oy1_ref, ox2_ref, oy2_ref):
    rows, lanes = sc_ref.shape
    sc0 = sc_ref[:]
    x1 = x1_ref[:]
    y1 = y1_ref[:]
    x2 = x2_ref[:]
    y2 = y2_ref[:]
    sc0 = jnp.where(sc0 > _CONF_THR, sc0, -1.0)
    a2 = (x2 - x1) * (y2 - y1)
    lane_iota = jax.lax.broadcasted_iota(jnp.int32, (rows, lanes), 1)
    col_iota = jax.lax.broadcasted_iota(jnp.int32, (rows, _OUT_LANES), 1)
    zeros_out = jnp.zeros((rows, _OUT_LANES), jnp.float32)

    def step(t, carry):
        sc, osc, ox1, oy1, ox2, oy2 = carry
        m = jnp.max(sc, axis=1, keepdims=True)
        keep = m > 0.0
        j = jnp.min(jnp.where(sc == m, lane_iota, jnp.int32(1 << 30)),
                    axis=1, keepdims=True)
        oh = lane_iota == j
        bx1 = jnp.sum(jnp.where(oh, x1, 0.0), axis=1, keepdims=True)
        by1 = jnp.sum(jnp.where(oh, y1, 0.0), axis=1, keepdims=True)
        bx2 = jnp.sum(jnp.where(oh, x2, 0.0), axis=1, keepdims=True)
        by2 = jnp.sum(jnp.where(oh, y2, 0.0), axis=1, keepdims=True)
        a1 = (bx2 - bx1) * (by2 - by1)
        ltx = jnp.maximum(bx1, x1)
        lty = jnp.maximum(by1, y1)
        rbx = jnp.minimum(bx2, x2)
        rby = jnp.minimum(by2, y2)
        w = jnp.maximum(rbx - ltx, 0.0)
        h = jnp.maximum(rby - lty, 0.0)
        inter = w * h
        iou = inter / (a1 + a2 - inter + 1e-8)
        supp = (iou > _IOU_THR) | oh
        sc = jnp.where(supp, -1.0, sc)
        col = col_iota == t
        osc = osc + jnp.where(col, jnp.where(keep, m, -1.0), 0.0)
        ox1 = ox1 + jnp.where(col, jnp.where(keep, bx1, 0.0), 0.0)
        oy1 = oy1 + jnp.where(col, jnp.where(keep, by1, 0.0), 0.0)
        ox2 = ox2 + jnp.where(col, jnp.where(keep, bx2, 0.0), 0.0)
        oy2 = oy2 + jnp.where(col, jnp.where(keep, by2, 0.0), 0.0)
        return sc, osc, ox1, oy1, ox2, oy2

    carry = (sc0, zeros_out, zeros_out, zeros_out, zeros_out, zeros_out)
    _, osc, ox1, oy1, ox2, oy2 = jax.lax.fori_loop(
        0, _MAX_PER_CLASS, step, carry)
    osc_ref[:] = osc
    ox1_ref[:] = ox1
    oy1_ref[:] = oy1
    ox2_ref[:] = ox2
    oy2_ref[:] = oy2


def _run_nms(scores, bx1, by1, bx2, by2):
    rows = scores.shape[0]
    out_sds = jax.ShapeDtypeStruct((rows, _OUT_LANES), jnp.float32)
    return pl.pallas_call(
        _nms_body,
        out_shape=(out_sds,) * 5,
    )(scores, bx1, by1, bx2, by2)


def _anchor_dims_np():
    import numpy as np
    ratios = [0.5, 1.0, 2.0]
    scales = [2.0 ** 0.0, 2.0 ** (1.0 / 3.0), 2.0 ** (2.0 / 3.0)]
    dims_all = []
    for area in [32.0 ** 2, 64.0 ** 2, 128.0 ** 2, 256.0 ** 2, 512.0 ** 2]:
        dims = []
        for r in ratios:
            h = np.sqrt(area / r)
            w = area / h
            for s in scales:
                dims.append([s * w, s * h])
        dims_all.append(np.array(dims, np.float32))
    return dims_all


def _get_anchors_np(H, W):
    import numpy as np
    strides = [2 ** i for i in range(3, 8)]
    dims_all = _anchor_dims_np()
    out = []
    for lvl in range(5):
        fh = int(np.ceil(H / strides[lvl]))
        fw = int(np.ceil(W / strides[lvl]))
        rx = (np.arange(fw, dtype=np.float32) + 0.5) * strides[lvl]
        ry = (np.arange(fh, dtype=np.float32) + 0.5) * strides[lvl]
        cx, cy = np.meshgrid(rx, ry)
        centers = np.tile(np.stack([cx, cy], -1)[:, :, None, :], [1, 1, 9, 1])
        dims = np.tile(dims_all[lvl][None, None, :, :], [fh, fw, 1, 1])
        out.append(np.concatenate([centers, dims], -1).reshape(-1, 4))
    return np.concatenate(out, 0)


@jax.jit
def kernel(images, predictions):
    H, W = images.shape[1], images.shape[2]
    anchors = jnp.asarray(_get_anchors_np(H, W))
    B = predictions.shape[0]

    box_preds = predictions[..., :4]
    cls_scores = jax.nn.sigmoid(predictions[..., 4:])
    sc_rows_full = jnp.transpose(cls_scores, (0, 2, 1)).reshape(_ROWS, _A)
    sc_rows_full = jnp.pad(sc_rows_full, ((0, 0), (0, _A_PAD - _A)))
    sc_rows_full = sc_rows_full.reshape(_ROWS, _A_PAD // 128, 128)

    table = jnp.concatenate(
        [box_preds, jnp.broadcast_to(anchors[None], (B, _A, 4))],
        axis=-1).reshape(B * _A, 8)

    cand_s, cand_i = _sc_select(sc_rows_full)

    cand_bp = table.at[cand_i.reshape(-1)].get(mode="promise_in_bounds")
    bp = cand_bp[..., :4].reshape(B, _NUM_CLASSES, _WIDTH, 4)
    an = cand_bp[..., 4:].reshape(B, _NUM_CLASSES, _WIDTH, 4)
    bvar = jnp.asarray([0.1, 0.1, 0.2, 0.2], jnp.float32)
    b = bp * bvar
    cxcy = b[..., :2] * an[..., 2:] + an[..., :2]
    wh = jnp.exp(b[..., 2:]) * an[..., 2:]
    boxes = jnp.concatenate([cxcy - wh / 2.0, cxcy + wh / 2.0], axis=-1)

    coords = boxes.reshape(_ROWS, _WIDTH, 4)
    bx1 = coords[:, :, 0]
    by1 = coords[:, :, 1]
    bx2 = coords[:, :, 2]
    by2 = coords[:, :, 3]

    osc, ox1, oy1, ox2, oy2 = _run_nms(cand_s, bx1, by1, bx2, by2)

    fs = osc[:, :_MAX_PER_CLASS].reshape(B, -1)
    fb = jnp.stack([ox1, oy1, ox2, oy2], axis=-1)[:, :_MAX_PER_CLASS, :]
    fb = fb.reshape(B, _NUM_CLASSES * _MAX_PER_CLASS, 4)

    ts, ti = jax.lax.top_k(fs, _MAX_DET)
    sel_b = jnp.take_along_axis(fb, ti[..., None], axis=1)
    sel_c = (ti // _MAX_PER_CLASS).astype(jnp.float32)
    mask = ts > 0.0
    ts_out = jnp.where(mask, ts, 0.0)
    sel_b = jnp.where(mask[..., None], sel_b, 0.0)
    sel_c = jnp.where(mask, sel_c, 0.0)
    valid = jnp.sum(mask.astype(jnp.int32), axis=1)
    return sel_b, ts_out, sel_c, valid


# speedup vs baseline: 6.6806x; 6.3122x over previous
import functools

import jax
import jax.numpy as jnp
from jax import lax
from jax.experimental import pallas as pl
from jax.experimental.pallas import tpu as pltpu
from jax.experimental.pallas import tpu_sc as plsc

_NUM_CLASSES = 80
_CONF_THR = 0.05
_IOU_THR = 0.5
_MAX_PER_CLASS = 100
_MAX_DET = 100
_PRE_TOPK = 1000

_A = 49104
_A_PAD = 49152
_ROWS = 320
_L = 16
_VECS = _A_PAD // _L
_GT_CAP = 1024
_EQ_CAP = 1024
_WIDTH = _GT_CAP + _EQ_CAP
_NWORKERS = 32
_ROWS_PER = _ROWS // _NWORKERS


def _sc_select_body(scores_hbm, table_hbm, cs_hbm, bp_hbm,
                    data_v, hist_v, bufs_v, bufi_v, rows_v, sem):
    wid = lax.axis_index("s") * 2 + lax.axis_index("c")
    lane = lax.iota(jnp.int32, _L)
    ones_i = jnp.ones((_L,), jnp.int32)

    def row_body(ri, _carry):
        r = wid * _ROWS_PER + ri
        img_base = (r // _NUM_CLASSES) * _A
        pltpu.sync_copy(scores_hbm.at[r], data_v)

        prefix = jnp.int32(0)
        c_above = jnp.int32(0)
        for p in range(4):
            shift = 24 - 8 * p

            def zero_body(i, c):
                hist_v[pl.ds(i * _L, _L)] = jnp.zeros((_L,), jnp.int32)
                return c
            lax.fori_loop(0, 256, zero_body, 0)

            def hist_body(i, c, _shift=shift, _prefix=prefix):
                v = data_v[i >> 3, pl.ds((i & 7) * _L, _L)]
                b = lax.bitcast_convert_type(v, jnp.int32)
                key = lax.shift_right_logical(b, _shift)
                binv = lax.bitwise_and(key, 0xFF)
                match = lax.shift_right_logical(key, 8) == _prefix
                idx = lax.bitwise_or(lax.shift_left(binv, 4), lane)
                plsc.addupdate_scatter(hist_v, [idx], ones_i, mask=match)
                return c
            lax.fori_loop(0, _VECS, hist_body, 0)

            def scan_body(i, carry, _c_above=c_above):
                cum, found_bin, c_add = carry
                bnum = 255 - i
                t = jnp.sum(hist_v[pl.ds(bnum * _L, _L)])
                not_found = found_bin < 0
                hit = not_found & (_c_above + cum + t >= _PRE_TOPK)
                found_bin = jnp.where(hit, bnum, found_bin)
                c_add = jnp.where(hit, cum, c_add)
                cum = jnp.where(not_found & jnp.logical_not(hit), cum + t, cum)
                return cum, found_bin, c_add
            _, fbin, c_add = lax.fori_loop(
                0, 256, scan_body, (jnp.int32(0), jnp.int32(-1), jnp.int32(0)))
            prefix = lax.bitwise_or(lax.shift_left(prefix, 8), fbin)
            c_above = c_above + c_add

        thr_bits = prefix
        need_ties = _PRE_TOPK - c_above

        def init_body(i, c):
            bufs_v[pl.ds(i * _L, _L)] = jnp.full((_L,), -1.0, jnp.float32)
            bufi_v[pl.ds(i * _L, _L)] = jnp.full((_L,), img_base, jnp.int32)
            return c
        lax.fori_loop(0, _WIDTH // _L, init_body, 0)

        def comp_body(i, carry):
            cgt, ceq = carry
            v = data_v[i >> 3, pl.ds((i & 7) * _L, _L)]
            b = lax.bitcast_convert_type(v, jnp.int32)
            gidx = lane + (img_base + i * _L)
            m_gt = b > thr_bits
            m_eq = b == thr_bits
            plsc.store_compressed(bufs_v.at[pl.ds(cgt, _L)], v, mask=m_gt)
            plsc.store_compressed(bufi_v.at[pl.ds(cgt, _L)], gidx, mask=m_gt)
            cgt = cgt + jnp.sum(m_gt.astype(jnp.int32))

            @pl.when(ceq <= _EQ_CAP - _L)
            def _():
                plsc.store_compressed(
                    bufs_v.at[pl.ds(_GT_CAP + ceq, _L)], v, mask=m_eq)
                plsc.store_compressed(
                    bufi_v.at[pl.ds(_GT_CAP + ceq, _L)], gidx, mask=m_eq)
            ceq = jnp.minimum(ceq + jnp.sum(m_eq.astype(jnp.int32)),
                              jnp.int32(_EQ_CAP))
            return cgt, ceq
        lax.fori_loop(0, _VECS, comp_body, (jnp.int32(0), jnp.int32(0)))

        def trim_body(i, c):
            pos = lane + i * _L
            v = bufs_v[pl.ds(_GT_CAP + i * _L, _L)]
            bufs_v[pl.ds(_GT_CAP + i * _L, _L)] = jnp.where(
                pos < need_ties, v, -1.0)
            return c
        lax.fori_loop(0, _EQ_CAP // _L, trim_body, 0)

        pltpu.async_copy(table_hbm.at[bufi_v], rows_v, sem).wait()

        pltpu.sync_copy(bufs_v, cs_hbm.at[r])
        pltpu.sync_copy(rows_v, bp_hbm.at[r])
        return _carry

    lax.fori_loop(0, _ROWS_PER, row_body, 0)


def _sc_select(scores_rows, table):
    mesh = plsc.VectorSubcoreMesh(core_axis_name="c", subcore_axis_name="s")
    return pl.kernel(
        _sc_select_body,
        mesh=mesh,
        out_type=(
            jax.ShapeDtypeStruct((_ROWS, _WIDTH), jnp.float32),
            jax.ShapeDtypeStruct((_ROWS, _WIDTH, 8), jnp.float32),
        ),
        scratch_types=[
            pltpu.VMEM((_A_PAD // 128, 128), jnp.float32),
            pltpu.VMEM((256 * _L,), jnp.int32),
            pltpu.VMEM((_WIDTH,), jnp.float32),
            pltpu.VMEM((_WIDTH,), jnp.int32),
            pltpu.VMEM((_WIDTH, 8), jnp.float32),
            pltpu.SemaphoreType.DMA,
        ],
        compiler_params=pltpu.CompilerParams(
            needs_layout_passes=False, use_tc_tiling_on_sc=False),
    )(scores_rows, table)


_OUT_LANES = 128


def _nms_body(sc_ref, x1_ref, y1_ref, x2_ref, y2_ref,
              osc_ref, ox1_ref, oy1_ref, ox2_ref, oy2_ref):
    rows, lanes = sc_ref.shape
    sc0 = sc_ref[:]
    x1 = x1_ref[:]
    y1 = y1_ref[:]
    x2 = x2_ref[:]
    y2 = y2_ref[:]
    sc0 = jnp.where(sc0 > _CONF_THR, sc0, -1.0)
    a2 = (x2 - x1) * (y2 - y1)
    lane_iota = jax.lax.broadcasted_iota(jnp.int32, (rows, lanes), 1)
    col_iota = jax.lax.broadcasted_iota(jnp.int32, (rows, _OUT_LANES), 1)
    zeros_out = jnp.zeros((rows, _OUT_LANES), jnp.float32)

    def step(t, carry):
        sc, osc, ox1, oy1, ox2, oy2 = carry
        m = jnp.max(sc, axis=1, keepdims=True)
        keep = m > 0.0
        j = jnp.min(jnp.where(sc == m, lane_iota, jnp.int32(1 << 30)),
                    axis=1, keepdims=True)
        oh = lane_iota == j
        bx1 = jnp.sum(jnp.where(oh, x1, 0.0), axis=1, keepdims=True)
        by1 = jnp.sum(jnp.where(oh, y1, 0.0), axis=1, keepdims=True)
        bx2 = jnp.sum(jnp.where(oh, x2, 0.0), axis=1, keepdims=True)
        by2 = jnp.sum(jnp.where(oh, y2, 0.0), axis=1, keepdims=True)
        a1 = (bx2 - bx1) * (by2 - by1)
        ltx = jnp.maximum(bx1, x1)
        lty = jnp.maximum(by1, y1)
        rbx = jnp.minimum(bx2, x2)
        rby = jnp.minimum(by2, y2)
        w = jnp.maximum(rbx - ltx, 0.0)
        h = jnp.maximum(rby - lty, 0.0)
        inter = w * h
        iou = inter / (a1 + a2 - inter + 1e-8)
        supp = (iou > _IOU_THR) | oh
        sc = jnp.where(supp, -1.0, sc)
        col = col_iota == t
        osc = osc + jnp.where(col, jnp.where(keep, m, -1.0), 0.0)
        ox1 = ox1 + jnp.where(col, jnp.where(keep, bx1, 0.0), 0.0)
        oy1 = oy1 + jnp.where(col, jnp.where(keep, by1, 0.0), 0.0)
        ox2 = ox2 + jnp.where(col, jnp.where(keep, bx2, 0.0), 0.0)
        oy2 = oy2 + jnp.where(col, jnp.where(keep, by2, 0.0), 0.0)
        return sc, osc, ox1, oy1, ox2, oy2

    carry = (sc0, zeros_out, zeros_out, zeros_out, zeros_out, zeros_out)
    _, osc, ox1, oy1, ox2, oy2 = jax.lax.fori_loop(
        0, _MAX_PER_CLASS, step, carry)
    osc_ref[:] = osc
    ox1_ref[:] = ox1
    oy1_ref[:] = oy1
    ox2_ref[:] = ox2
    oy2_ref[:] = oy2


def _run_nms(scores, bx1, by1, bx2, by2):
    rows = scores.shape[0]
    out_sds = jax.ShapeDtypeStruct((rows, _OUT_LANES), jnp.float32)
    return pl.pallas_call(
        _nms_body,
        out_shape=(out_sds,) * 5,
    )(scores, bx1, by1, bx2, by2)


def _anchor_dims_np():
    import numpy as np
    ratios = [0.5, 1.0, 2.0]
    scales = [2.0 ** 0.0, 2.0 ** (1.0 / 3.0), 2.0 ** (2.0 / 3.0)]
    dims_all = []
    for area in [32.0 ** 2, 64.0 ** 2, 128.0 ** 2, 256.0 ** 2, 512.0 ** 2]:
        dims = []
        for r in ratios:
            h = np.sqrt(area / r)
            w = area / h
            for s in scales:
                dims.append([s * w, s * h])
        dims_all.append(np.array(dims, np.float32))
    return dims_all


def _get_anchors_np(H, W):
    import numpy as np
    strides = [2 ** i for i in range(3, 8)]
    dims_all = _anchor_dims_np()
    out = []
    for lvl in range(5):
        fh = int(np.ceil(H / strides[lvl]))
        fw = int(np.ceil(W / strides[lvl]))
        rx = (np.arange(fw, dtype=np.float32) + 0.5) * strides[lvl]
        ry = (np.arange(fh, dtype=np.float32) + 0.5) * strides[lvl]
        cx, cy = np.meshgrid(rx, ry)
        centers = np.tile(np.stack([cx, cy], -1)[:, :, None, :], [1, 1, 9, 1])
        dims = np.tile(dims_all[lvl][None, None, :, :], [fh, fw, 1, 1])
        out.append(np.concatenate([centers, dims], -1).reshape(-1, 4))
    return np.concatenate(out, 0)


@jax.jit
def kernel(images, predictions):
    H, W = images.shape[1], images.shape[2]
    anchors = jnp.asarray(_get_anchors_np(H, W))
    B = predictions.shape[0]

    box_preds = predictions[..., :4]
    cls_scores = jax.nn.sigmoid(predictions[..., 4:])
    sc_rows_full = jnp.transpose(cls_scores, (0, 2, 1)).reshape(_ROWS, _A)
    sc_rows_full = jnp.pad(sc_rows_full, ((0, 0), (0, _A_PAD - _A)))
    sc_rows_full = sc_rows_full.reshape(_ROWS, _A_PAD // 128, 128)

    table = jnp.concatenate(
        [box_preds, jnp.broadcast_to(anchors[None], (B, _A, 4))],
        axis=-1).reshape(B * _A, 8)

    cand_s, cand_bp = _sc_select(sc_rows_full, table)
    bp = cand_bp[..., :4].reshape(B, _NUM_CLASSES, _WIDTH, 4)
    an = cand_bp[..., 4:].reshape(B, _NUM_CLASSES, _WIDTH, 4)
    bvar = jnp.asarray([0.1, 0.1, 0.2, 0.2], jnp.float32)
    b = bp * bvar
    cxcy = b[..., :2] * an[..., 2:] + an[..., :2]
    wh = jnp.exp(b[..., 2:]) * an[..., 2:]
    boxes = jnp.concatenate([cxcy - wh / 2.0, cxcy + wh / 2.0], axis=-1)

    coords = boxes.reshape(_ROWS, _WIDTH, 4)
    bx1 = coords[:, :, 0]
    by1 = coords[:, :, 1]
    bx2 = coords[:, :, 2]
    by2 = coords[:, :, 3]

    osc, ox1, oy1, ox2, oy2 = _run_nms(cand_s, bx1, by1, bx2, by2)

    fs = osc[:, :_MAX_PER_CLASS].reshape(B, -1)
    fb = jnp.stack([ox1, oy1, ox2, oy2], axis=-1)[:, :_MAX_PER_CLASS, :]
    fb = fb.reshape(B, _NUM_CLASSES * _MAX_PER_CLASS, 4)

    ts, ti = jax.lax.top_k(fs, _MAX_DET)
    sel_b = jnp.take_along_axis(fb, ti[..., None], axis=1)
    sel_c = (ti // _MAX_PER_CLASS).astype(jnp.float32)
    mask = ts > 0.0
    ts_out = jnp.where(mask, ts, 0.0)
    sel_b = jnp.where(mask[..., None], sel_b, 0.0)
    sel_c = jnp.where(mask, sel_c, 0.0)
    valid = jnp.sum(mask.astype(jnp.int32), axis=1)
    return sel_b, ts_out, sel_c, valid


# speedup vs baseline: 9.5227x; 1.4254x over previous
import functools

import jax
import jax.numpy as jnp
from jax import lax
from jax.experimental import pallas as pl
from jax.experimental.pallas import tpu as pltpu
from jax.experimental.pallas import tpu_sc as plsc

_NUM_CLASSES = 80
_CONF_THR = 0.05
_IOU_THR = 0.5
_MAX_PER_CLASS = 100
_MAX_DET = 100
_PRE_TOPK = 1000

_A = 49104
_A_PAD = 49152
_ROWS = 320
_L = 16
_VECS = _A_PAD // _L
_GT_CAP = 1024
_EQ_CAP = 1024
_WIDTH = _GT_CAP + _EQ_CAP
_NWORKERS = 32
_ROWS_PER = _ROWS // _NWORKERS


def _sc_select_body(scores_hbm, table_hbm, cs_hbm, bp_hbm,
                    data_v, hist_v, bufs_v, bufi_v, rows_v, sem):
    wid = lax.axis_index("s") * 2 + lax.axis_index("c")
    lane = lax.iota(jnp.int32, _L)
    ones_i = jnp.ones((_L,), jnp.int32)

    def row_body(ri, _carry):
        r = wid * _ROWS_PER + ri
        img_base = (r // _NUM_CLASSES) * _A
        pltpu.sync_copy(scores_hbm.at[r], data_v)

        prefix = jnp.int32(0)
        c_above = jnp.int32(0)
        for p in range(4):
            shift = 24 - 8 * p

            def zero_body(i, c):
                hist_v[pl.ds(i * _L, _L)] = jnp.zeros((_L,), jnp.int32)
                return c
            lax.fori_loop(0, 256, zero_body, 0)

            @plsc.parallel_loop(0, _VECS, unroll=8)
            def hist_body(i, _shift=shift, _prefix=prefix):
                v = data_v[i >> 3, pl.ds((i & 7) * _L, _L)]
                b = lax.bitcast_convert_type(v, jnp.int32)
                key = lax.shift_right_logical(b, _shift)
                binv = lax.bitwise_and(key, 0xFF)
                match = lax.shift_right_logical(key, 8) == _prefix
                idx = lax.bitwise_or(lax.shift_left(binv, 4), lane)
                plsc.addupdate_scatter(hist_v, [idx], ones_i, mask=match)

            def scan_body(i, carry, _c_above=c_above):
                cum, found_bin, c_add = carry
                bnum = 255 - i
                t = jnp.sum(hist_v[pl.ds(bnum * _L, _L)])
                not_found = found_bin < 0
                hit = not_found & (_c_above + cum + t >= _PRE_TOPK)
                found_bin = jnp.where(hit, bnum, found_bin)
                c_add = jnp.where(hit, cum, c_add)
                cum = jnp.where(not_found & jnp.logical_not(hit), cum + t, cum)
                return cum, found_bin, c_add
            _, fbin, c_add = lax.fori_loop(
                0, 256, scan_body, (jnp.int32(0), jnp.int32(-1), jnp.int32(0)))
            prefix = lax.bitwise_or(lax.shift_left(prefix, 8), fbin)
            c_above = c_above + c_add

        thr_bits = prefix
        need_ties = _PRE_TOPK - c_above

        def init_body(i, c):
            bufs_v[pl.ds(i * _L, _L)] = jnp.full((_L,), -1.0, jnp.float32)
            bufi_v[pl.ds(i * _L, _L)] = jnp.full((_L,), img_base, jnp.int32)
            return c
        lax.fori_loop(0, _WIDTH // _L, init_body, 0)

        def comp_body(i, carry):
            cgt, ceq = carry
            v = data_v[i >> 3, pl.ds((i & 7) * _L, _L)]
            b = lax.bitcast_convert_type(v, jnp.int32)
            gidx = lane + (img_base + i * _L)
            m_gt = b > thr_bits
            m_eq = b == thr_bits
            plsc.store_compressed(bufs_v.at[pl.ds(cgt, _L)], v, mask=m_gt)
            plsc.store_compressed(bufi_v.at[pl.ds(cgt, _L)], gidx, mask=m_gt)
            cgt = cgt + jnp.sum(m_gt.astype(jnp.int32))

            @pl.when(ceq <= _EQ_CAP - _L)
            def _():
                plsc.store_compressed(
                    bufs_v.at[pl.ds(_GT_CAP + ceq, _L)], v, mask=m_eq)
                plsc.store_compressed(
                    bufi_v.at[pl.ds(_GT_CAP + ceq, _L)], gidx, mask=m_eq)
            ceq = jnp.minimum(ceq + jnp.sum(m_eq.astype(jnp.int32)),
                              jnp.int32(_EQ_CAP))
            return cgt, ceq
        plsc.parallel_loop(
            0, _VECS, unroll=4,
            carry=(jnp.int32(0), jnp.int32(0)))(comp_body)

        def trim_body(i, c):
            pos = lane + i * _L
            v = bufs_v[pl.ds(_GT_CAP + i * _L, _L)]
            bufs_v[pl.ds(_GT_CAP + i * _L, _L)] = jnp.where(
                pos < need_ties, v, -1.0)
            return c
        lax.fori_loop(0, _EQ_CAP // _L, trim_body, 0)

        pltpu.async_copy(table_hbm.at[bufi_v], rows_v, sem).wait()

        pltpu.sync_copy(bufs_v, cs_hbm.at[r])
        pltpu.sync_copy(rows_v, bp_hbm.at[r])
        return _carry

    lax.fori_loop(0, _ROWS_PER, row_body, 0)


def _sc_select(scores_rows, table):
    mesh = plsc.VectorSubcoreMesh(core_axis_name="c", subcore_axis_name="s")
    return pl.kernel(
        _sc_select_body,
        mesh=mesh,
        out_type=(
            jax.ShapeDtypeStruct((_ROWS, _WIDTH), jnp.float32),
            jax.ShapeDtypeStruct((_ROWS, _WIDTH, 8), jnp.float32),
        ),
        scratch_types=[
            pltpu.VMEM((_A_PAD // 128, 128), jnp.float32),
            pltpu.VMEM((256 * _L,), jnp.int32),
            pltpu.VMEM((_WIDTH,), jnp.float32),
            pltpu.VMEM((_WIDTH,), jnp.int32),
            pltpu.VMEM((_WIDTH, 8), jnp.float32),
            pltpu.SemaphoreType.DMA,
        ],
        compiler_params=pltpu.CompilerParams(
            needs_layout_passes=False, use_tc_tiling_on_sc=False),
    )(scores_rows, table)


_OUT_LANES = 128


def _nms_body(sc_ref, x1_ref, y1_ref, x2_ref, y2_ref,
              osc_ref, ox1_ref, oy1_ref, ox2_ref, oy2_ref):
    rows, lanes = sc_ref.shape
    sc0 = sc_ref[:]
    x1 = x1_ref[:]
    y1 = y1_ref[:]
    x2 = x2_ref[:]
    y2 = y2_ref[:]
    sc0 = jnp.where(sc0 > _CONF_THR, sc0, -1.0)
    a2 = (x2 - x1) * (y2 - y1)
    lane_iota = jax.lax.broadcasted_iota(jnp.int32, (rows, lanes), 1)
    col_iota = jax.lax.broadcasted_iota(jnp.int32, (rows, _OUT_LANES), 1)
    zeros_out = jnp.zeros((rows, _OUT_LANES), jnp.float32)

    def step(t, carry):
        sc, osc, ox1, oy1, ox2, oy2 = carry
        m = jnp.max(sc, axis=1, keepdims=True)
        keep = m > 0.0
        j = jnp.min(jnp.where(sc == m, lane_iota, jnp.int32(1 << 30)),
                    axis=1, keepdims=True)
        oh = lane_iota == j
        bx1 = jnp.sum(jnp.where(oh, x1, 0.0), axis=1, keepdims=True)
        by1 = jnp.sum(jnp.where(oh, y1, 0.0), axis=1, keepdims=True)
        bx2 = jnp.sum(jnp.where(oh, x2, 0.0), axis=1, keepdims=True)
        by2 = jnp.sum(jnp.where(oh, y2, 0.0), axis=1, keepdims=True)
        a1 = (bx2 - bx1) * (by2 - by1)
        ltx = jnp.maximum(bx1, x1)
        lty = jnp.maximum(by1, y1)
        rbx = jnp.minimum(bx2, x2)
        rby = jnp.minimum(by2, y2)
        w = jnp.maximum(rbx - ltx, 0.0)
        h = jnp.maximum(rby - lty, 0.0)
        inter = w * h
        iou = inter / (a1 + a2 - inter + 1e-8)
        supp = (iou > _IOU_THR) | oh
        sc = jnp.where(supp, -1.0, sc)
        col = col_iota == t
        osc = osc + jnp.where(col, jnp.where(keep, m, -1.0), 0.0)
        ox1 = ox1 + jnp.where(col, jnp.where(keep, bx1, 0.0), 0.0)
        oy1 = oy1 + jnp.where(col, jnp.where(keep, by1, 0.0), 0.0)
        ox2 = ox2 + jnp.where(col, jnp.where(keep, bx2, 0.0), 0.0)
        oy2 = oy2 + jnp.where(col, jnp.where(keep, by2, 0.0), 0.0)
        return sc, osc, ox1, oy1, ox2, oy2

    carry = (sc0, zeros_out, zeros_out, zeros_out, zeros_out, zeros_out)
    _, osc, ox1, oy1, ox2, oy2 = jax.lax.fori_loop(
        0, _MAX_PER_CLASS, step, carry)
    osc_ref[:] = osc
    ox1_ref[:] = ox1
    oy1_ref[:] = oy1
    ox2_ref[:] = ox2
    oy2_ref[:] = oy2


def _run_nms(scores, bx1, by1, bx2, by2):
    rows = scores.shape[0]
    out_sds = jax.ShapeDtypeStruct((rows, _OUT_LANES), jnp.float32)
    return pl.pallas_call(
        _nms_body,
        out_shape=(out_sds,) * 5,
    )(scores, bx1, by1, bx2, by2)


def _anchor_dims_np():
    import numpy as np
    ratios = [0.5, 1.0, 2.0]
    scales = [2.0 ** 0.0, 2.0 ** (1.0 / 3.0), 2.0 ** (2.0 / 3.0)]
    dims_all = []
    for area in [32.0 ** 2, 64.0 ** 2, 128.0 ** 2, 256.0 ** 2, 512.0 ** 2]:
        dims = []
        for r in ratios:
            h = np.sqrt(area / r)
            w = area / h
            for s in scales:
                dims.append([s * w, s * h])
        dims_all.append(np.array(dims, np.float32))
    return dims_all


def _get_anchors_np(H, W):
    import numpy as np
    strides = [2 ** i for i in range(3, 8)]
    dims_all = _anchor_dims_np()
    out = []
    for lvl in range(5):
        fh = int(np.ceil(H / strides[lvl]))
        fw = int(np.ceil(W / strides[lvl]))
        rx = (np.arange(fw, dtype=np.float32) + 0.5) * strides[lvl]
        ry = (np.arange(fh, dtype=np.float32) + 0.5) * strides[lvl]
        cx, cy = np.meshgrid(rx, ry)
        centers = np.tile(np.stack([cx, cy], -1)[:, :, None, :], [1, 1, 9, 1])
        dims = np.tile(dims_all[lvl][None, None, :, :], [fh, fw, 1, 1])
        out.append(np.concatenate([centers, dims], -1).reshape(-1, 4))
    return np.concatenate(out, 0)


@jax.jit
def kernel(images, predictions):
    H, W = images.shape[1], images.shape[2]
    anchors = jnp.asarray(_get_anchors_np(H, W))
    B = predictions.shape[0]

    box_preds = predictions[..., :4]
    cls_scores = jax.nn.sigmoid(predictions[..., 4:])
    sc_rows_full = jnp.transpose(cls_scores, (0, 2, 1)).reshape(_ROWS, _A)
    sc_rows_full = jnp.pad(sc_rows_full, ((0, 0), (0, _A_PAD - _A)))
    sc_rows_full = sc_rows_full.reshape(_ROWS, _A_PAD // 128, 128)

    table = jnp.concatenate(
        [box_preds, jnp.broadcast_to(anchors[None], (B, _A, 4))],
        axis=-1).reshape(B * _A, 8)

    cand_s, cand_bp = _sc_select(sc_rows_full, table)
    bp = cand_bp[..., :4].reshape(B, _NUM_CLASSES, _WIDTH, 4)
    an = cand_bp[..., 4:].reshape(B, _NUM_CLASSES, _WIDTH, 4)
    bvar = jnp.asarray([0.1, 0.1, 0.2, 0.2], jnp.float32)
    b = bp * bvar
    cxcy = b[..., :2] * an[..., 2:] + an[..., :2]
    wh = jnp.exp(b[..., 2:]) * an[..., 2:]
    boxes = jnp.concatenate([cxcy - wh / 2.0, cxcy + wh / 2.0], axis=-1)

    coords = boxes.reshape(_ROWS, _WIDTH, 4)
    bx1 = coords[:, :, 0]
    by1 = coords[:, :, 1]
    bx2 = coords[:, :, 2]
    by2 = coords[:, :, 3]

    osc, ox1, oy1, ox2, oy2 = _run_nms(cand_s, bx1, by1, bx2, by2)

    fs = osc[:, :_MAX_PER_CLASS].reshape(B, -1)
    fb = jnp.stack([ox1, oy1, ox2, oy2], axis=-1)[:, :_MAX_PER_CLASS, :]
    fb = fb.reshape(B, _NUM_CLASSES * _MAX_PER_CLASS, 4)

    ts, ti = jax.lax.top_k(fs, _MAX_DET)
    sel_b = jnp.take_along_axis(fb, ti[..., None], axis=1)
    sel_c = (ti // _MAX_PER_CLASS).astype(jnp.float32)
    mask = ts > 0.0
    ts_out = jnp.where(mask, ts, 0.0)
    sel_b = jnp.where(mask[..., None], sel_b, 0.0)
    sel_c = jnp.where(mask, sel_c, 0.0)
    valid = jnp.sum(mask.astype(jnp.int32), axis=1)
    return sel_b, ts_out, sel_c, valid
